# baseline (speedup 1.0000x reference)
import functools

import jax
import jax.numpy as jnp
from jax import lax
from jax.experimental import pallas as pl
from jax.experimental.pallas import tpu as pltpu
from jax.experimental.pallas import tpu_sc as plsc

VOCAB_ = 1000000
DIM_ = 64
PAD_ = 0
BATCH_ = 4096
SEQ_ = 200


_BR = 16384
_NBLK = (VOCAB_ + _BR - 1) // _BR


def _mass_body(t_ref, w1_ref, b1_ref, w2t_ref, b2_ref, o_ref):
    t = t_ref[...]
    h = jnp.tanh(
        jnp.dot(t, w1_ref[...], preferred_element_type=jnp.float32)
        + b1_ref[...]
    )
    raw = jnp.sum(h * w2t_ref[...], axis=1) + b2_ref[0, 0]
    o_ref[...] = raw.reshape(1, 1, _BR)


def _mass_tc(table, W1, b1, W2, b2):
    w2t = W2.reshape(1, DIM_ // 4)
    b1r = b1.reshape(1, DIM_ // 4)
    b2r = b2.reshape(1, 1)
    out = pl.pallas_call(
        _mass_body,
        grid=(_NBLK,),
        in_specs=[
            pl.BlockSpec((_BR, DIM_), lambda i: (i, 0)),
            pl.BlockSpec((DIM_, DIM_ // 4), lambda i: (0, 0)),
            pl.BlockSpec((1, DIM_ // 4), lambda i: (0, 0)),
            pl.BlockSpec((1, DIM_ // 4), lambda i: (0, 0)),
            pl.BlockSpec((1, 1), lambda i: (0, 0)),
        ],
        out_specs=pl.BlockSpec((1, 1, _BR), lambda i: (i, 0, 0)),
        out_shape=jax.ShapeDtypeStruct((_NBLK, 1, _BR), jnp.float32),
    )(table, W1, b1r, w2t, b2r)
    return out.reshape(_NBLK * _BR)


_NW = 32
_RPT = BATCH_ // _NW
_HS = 112
_NL = 16


def _pool_body(tok_hbm, mass_hbm, table_hbm, out_hbm,
               tok2, mass_v, rows_v, out_buf, sem):
    info = plsc.get_sparse_core_info()
    nc = info.num_cores
    wid = lax.axis_index("s") * nc + lax.axis_index("c")
    base_row = wid * _RPT

    zeros16 = jnp.zeros((_NL,), jnp.int32)
    tok2[1, pl.ds(80, _NL)] = zeros16
    tok2[1, pl.ds(96, _NL)] = zeros16

    def row_body(r, _):
        row = base_row + r
        pltpu.sync_copy(tok_hbm.at[pl.ds(row * SEQ_, _HS)],
                        tok2.at[0])
        pltpu.sync_copy(tok_hbm.at[pl.ds(row * SEQ_ + _HS, SEQ_ - _HS)],
                        tok2.at[1, pl.ds(0, SEQ_ - _HS)])
        pltpu.async_copy(mass_hbm.at[tok2.at[0]],
                         mass_v.at[pl.ds(0, _HS)], sem).wait()
        pltpu.async_copy(mass_hbm.at[tok2.at[1]],
                         mass_v.at[pl.ds(_HS, _HS)], sem).wait()
        pltpu.async_copy(table_hbm.at[tok2.at[0]],
                         rows_v.at[pl.ds(0, _HS)], sem).wait()
        pltpu.async_copy(table_hbm.at[tok2.at[1]],
                         rows_v.at[pl.ds(_HS, _HS)], sem).wait()

        lane = lax.broadcasted_iota(jnp.int32, (_NL,), 0)
        zero = jnp.zeros((_NL,), jnp.float32)

        for c in range(13):
            if c < 7:
                tk = tok2[0, pl.ds(c * _NL, _NL)]
            else:
                tk = tok2[1, pl.ds(c * _NL - _HS, _NL)]
            m = mass_v[pl.ds(c * _NL, _NL)]
            cond = tk == PAD_
            if c == 12:
                cond = jnp.logical_or(cond, lane >= 8)
            e = jnp.where(cond, zero, jnp.exp(m))
            mass_v[pl.ds(c * _NL, _NL)] = e

        def acc_body(c, accs):
            a0, a1, a2, a3 = accs
            wv = mass_v[pl.ds(c * _NL, _NL)]
            for k in range(_NL):
                s = c * _NL + k
                w = wv[k]
                a0 = a0 + w * rows_v[s, pl.ds(0, _NL)]
                a1 = a1 + w * rows_v[s, pl.ds(_NL, _NL)]
                a2 = a2 + w * rows_v[s, pl.ds(2 * _NL, _NL)]
                a3 = a3 + w * rows_v[s, pl.ds(3 * _NL, _NL)]
            return (a0, a1, a2, a3)

        z = jnp.zeros((_NL,), jnp.float32)
        a0, a1, a2, a3 = lax.fori_loop(0, 13, acc_body, (z, z, z, z))
        out_buf[r, pl.ds(0, _NL)] = a0
        out_buf[r, pl.ds(_NL, _NL)] = a1
        out_buf[r, pl.ds(2 * _NL, _NL)] = a2
        out_buf[r, pl.ds(3 * _NL, _NL)] = a3
        return _

    lax.fori_loop(0, _RPT, row_body, 0)
    pltpu.sync_copy(out_buf, out_hbm.at[pl.ds(base_row, _RPT)])


def _pool_sc(tok_flat, mass, table):
    mesh = plsc.VectorSubcoreMesh(core_axis_name="c", subcore_axis_name="s")
    f = functools.partial(
        pl.kernel,
        out_type=jax.ShapeDtypeStruct((BATCH_, DIM_), jnp.float32),
        mesh=mesh,
        scratch_types=[
            pltpu.VMEM((2, _HS), jnp.int32),
            pltpu.VMEM((2 * _HS,), jnp.float32),
            pltpu.VMEM((2 * _HS, DIM_), jnp.float32),
            pltpu.VMEM((_RPT, DIM_), jnp.float32),
            pltpu.SemaphoreType.DMA,
        ],
        compiler_params=pltpu.CompilerParams(use_tc_tiling_on_sc=False),
    )(_pool_body)
    return f(tok_flat, mass, table)


def _norm_body(x_ref, o_ref):
    x = x_ref[...]
    n = jnp.sqrt(jnp.sum(x * x, axis=1, keepdims=True))
    o_ref[...] = x / jnp.maximum(n, 1e-12)


def _normalize_tc(sv):
    return pl.pallas_call(
        _norm_body,
        out_shape=jax.ShapeDtypeStruct((BATCH_, DIM_), jnp.float32),
    )(sv)


def kernel(token_ids, table, W1, b1, W2, b2):
    mass = _mass_tc(table, W1, b1, W2, b2)
    sv = _pool_sc(token_ids.reshape(-1), mass, table)
    return _normalize_tc(sv)


# speedup vs baseline: 2.0577x; 2.0577x over previous
import functools

import jax
import jax.numpy as jnp
from jax import lax
from jax.experimental import pallas as pl
from jax.experimental.pallas import tpu as pltpu
from jax.experimental.pallas import tpu_sc as plsc

VOCAB_ = 1000000
DIM_ = 64
HID_ = 16
PAD_ = 0
BATCH_ = 4096
SEQ_ = 200


_BR = 16384
_NBLK = (VOCAB_ + _BR - 1) // _BR


def _mass_body(t_ref, w1t_ref, b1_ref, w2_ref, b2_ref, o_ref):
    t = t_ref[...]
    x = jnp.dot(w1t_ref[...], t, preferred_element_type=jnp.float32)
    x = x + b1_ref[...]
    e2 = jnp.exp(2.0 * x)
    h = 1.0 - 2.0 / (e2 + 1.0)
    raw = jnp.sum(h * w2_ref[...], axis=0) + b2_ref[0, 0]
    o_ref[...] = raw.reshape(1, 1, _BR)


def _mass_tc(table_t, W1, b1, W2, b2):
    w1t = W1.T
    b1r = b1.reshape(HID_, 1)
    b2r = b2.reshape(1, 1)
    out = pl.pallas_call(
        _mass_body,
        grid=(_NBLK,),
        in_specs=[
            pl.BlockSpec((DIM_, _BR), lambda i: (0, i)),
            pl.BlockSpec((HID_, DIM_), lambda i: (0, 0)),
            pl.BlockSpec((HID_, 1), lambda i: (0, 0)),
            pl.BlockSpec((HID_, 1), lambda i: (0, 0)),
            pl.BlockSpec((1, 1), lambda i: (0, 0)),
        ],
        out_specs=pl.BlockSpec((1, 1, _BR), lambda i: (i, 0, 0)),
        out_shape=jax.ShapeDtypeStruct((_NBLK, 1, _BR), jnp.float32),
    )(table_t, w1t, b1r, W2, b2r)
    return out.reshape(_NBLK * _BR)


_NW = 32
_RPT = BATCH_ // _NW
_SP = 208
_NL = 16
_NB = 4


def _pool_body(tok_hbm, mass_hbm, table_hbm, out_hbm,
               tok_v, msv, rows0, rows1, rows2, rows3, out_buf,
               sem_t, sem_m, sem0, sem1, sem2, sem3):
    info = plsc.get_sparse_core_info()
    nc = info.num_cores
    wid = lax.axis_index("s") * nc + lax.axis_index("c")
    base_row = wid * _RPT
    rows = (rows0, rows1, rows2, rows3)
    sems = (sem0, sem1, sem2, sem3)

    z16 = jnp.zeros((_NL,), jnp.int32)

    def zero_body(r, _):
        tok_v[r, pl.ds(192, _NL)] = z16
        return _
    lax.fori_loop(0, _RPT, zero_body, 0)

    pltpu.async_copy(tok_hbm.at[pl.ds(base_row, _RPT), :],
                     tok_v.at[:, pl.ds(0, SEQ_)], sem_t).wait()

    def mass_fire(r, _):
        pltpu.async_copy(mass_hbm.at[tok_v.at[r, pl.ds(0, 112)]],
                         msv.at[r, pl.ds(0, 112)], sem_m)
        pltpu.async_copy(mass_hbm.at[tok_v.at[r, pl.ds(112, 96)]],
                         msv.at[r, pl.ds(112, 96)], sem_m)
        return _
    lax.fori_loop(0, _RPT, mass_fire, 0)

    def mass_drain(r, _):
        pltpu.make_async_copy(mass_hbm.at[tok_v.at[r, pl.ds(0, 112)]],
                              msv.at[r, pl.ds(0, 112)], sem_m).wait()
        pltpu.make_async_copy(mass_hbm.at[tok_v.at[r, pl.ds(112, 96)]],
                              msv.at[r, pl.ds(112, 96)], sem_m).wait()
        return _
    lax.fori_loop(0, _RPT, mass_drain, 0)

    def fire_rows(r, slot, sem):
        pltpu.async_copy(table_hbm.at[tok_v.at[r, pl.ds(0, 112)]],
                         slot.at[pl.ds(0, 112)], sem)
        pltpu.async_copy(table_hbm.at[tok_v.at[r, pl.ds(112, 96)]],
                         slot.at[pl.ds(112, 96)], sem)

    def wait_rows(r, slot, sem):
        pltpu.make_async_copy(table_hbm.at[tok_v.at[r, pl.ds(0, 112)]],
                              slot.at[pl.ds(0, 112)], sem).wait()
        pltpu.make_async_copy(table_hbm.at[tok_v.at[r, pl.ds(112, 96)]],
                              slot.at[pl.ds(112, 96)], sem).wait()

    for b in range(_NB):
        fire_rows(b, rows[b], sems[b])

    zf = jnp.zeros((_NL,), jnp.float32)

    def group_body(g, _):
        for b in range(_NB):
            r = g * _NB + b
            slot, sem = rows[b], sems[b]
            wait_rows(r, slot, sem)

            def acc_body(c, accs):
                a0, a1, a2, a3 = accs
                tk = tok_v[r, pl.ds(c * _NL, _NL)]
                m = msv[r, pl.ds(c * _NL, _NL)]
                wv = jnp.where(tk == PAD_, zf, jnp.exp(m))
                base = c * _NL
                for k in range(_NL):
                    s = base + k
                    w = wv[k]
                    a0 = a0 + w * slot[s, pl.ds(0, _NL)]
                    a1 = a1 + w * slot[s, pl.ds(_NL, _NL)]
                    a2 = a2 + w * slot[s, pl.ds(2 * _NL, _NL)]
                    a3 = a3 + w * slot[s, pl.ds(3 * _NL, _NL)]
                return (a0, a1, a2, a3)

            a0, a1, a2, a3 = lax.fori_loop(0, 13, acc_body, (zf, zf, zf, zf))
            out_buf[r, pl.ds(0, _NL)] = a0
            out_buf[r, pl.ds(_NL, _NL)] = a1
            out_buf[r, pl.ds(2 * _NL, _NL)] = a2
            out_buf[r, pl.ds(3 * _NL, _NL)] = a3

            @pl.when(g < (_RPT // _NB) - 1)
            def _fire_next():
                fire_rows(r + _NB, slot, sem)
        return _

    lax.fori_loop(0, _RPT // _NB, group_body, 0)
    pltpu.sync_copy(out_buf, out_hbm.at[pl.ds(base_row, _RPT)])


def _pool_sc(token_ids, mass, table):
    mesh = plsc.VectorSubcoreMesh(core_axis_name="c", subcore_axis_name="s")
    f = functools.partial(
        pl.kernel,
        out_type=jax.ShapeDtypeStruct((BATCH_, DIM_), jnp.float32),
        mesh=mesh,
        scratch_types=[
            pltpu.VMEM((_RPT, _SP), jnp.int32),
            pltpu.VMEM((_RPT, _SP), jnp.float32),
            pltpu.VMEM((_SP, DIM_), jnp.float32),
            pltpu.VMEM((_SP, DIM_), jnp.float32),
            pltpu.VMEM((_SP, DIM_), jnp.float32),
            pltpu.VMEM((_SP, DIM_), jnp.float32),
            pltpu.VMEM((_RPT, DIM_), jnp.float32),
            pltpu.SemaphoreType.DMA,
            pltpu.SemaphoreType.DMA,
            pltpu.SemaphoreType.DMA,
            pltpu.SemaphoreType.DMA,
            pltpu.SemaphoreType.DMA,
            pltpu.SemaphoreType.DMA,
        ],
        compiler_params=pltpu.CompilerParams(use_tc_tiling_on_sc=False),
    )(_pool_body)
    return f(token_ids, mass, table)


def _norm_body(x_ref, o_ref):
    x = x_ref[...]
    n = jnp.sqrt(jnp.sum(x * x, axis=1, keepdims=True))
    o_ref[...] = x / jnp.maximum(n, 1e-12)


def _normalize_tc(sv):
    return pl.pallas_call(
        _norm_body,
        out_shape=jax.ShapeDtypeStruct((BATCH_, DIM_), jnp.float32),
    )(sv)


def kernel(token_ids, table, W1, b1, W2, b2):
    mass = _mass_tc(table.T, W1, b1, W2, b2)
    sv = _pool_sc(token_ids, mass, table)
    return _normalize_tc(sv)


# speedup vs baseline: 6.5092x; 3.1634x over previous
import functools

import jax
import jax.numpy as jnp
from jax import lax
from jax.experimental import pallas as pl
from jax.experimental.pallas import tpu as pltpu
from jax.experimental.pallas import tpu_sc as plsc

VOCAB_ = 1000000
DIM_ = 64
HID_ = 16
PAD_ = 0
BATCH_ = 4096
SEQ_ = 200
LINE_ = 128


_BR = 16384
_NBLK = (VOCAB_ + _BR - 1) // _BR


def _mass_body(t_ref, w1t_ref, b1_ref, w2_ref, b2_ref, o_ref, o2_ref):
    t = t_ref[...]
    x = jnp.dot(w1t_ref[...], t, preferred_element_type=jnp.float32)
    x = x + b1_ref[...]
    e2 = jnp.exp(2.0 * x)
    h = 1.0 - 2.0 / (e2 + 1.0)
    raw = jnp.sum(h * w2_ref[...], axis=0) + b2_ref[0, 0]
    o_ref[...] = raw.reshape(1, 1, _BR)
    o2_ref[:, :DIM_] = t.T
    o2_ref[:, DIM_:] = jnp.zeros((_BR, LINE_ - DIM_), jnp.float32)


def _mass_tc(table_t, W1, b1, W2, b2):
    w1t = W1.T
    b1r = b1.reshape(HID_, 1)
    b2r = b2.reshape(1, 1)
    out, taug = pl.pallas_call(
        _mass_body,
        grid=(_NBLK,),
        in_specs=[
            pl.BlockSpec((DIM_, _BR), lambda i: (0, i)),
            pl.BlockSpec((HID_, DIM_), lambda i: (0, 0)),
            pl.BlockSpec((HID_, 1), lambda i: (0, 0)),
            pl.BlockSpec((HID_, 1), lambda i: (0, 0)),
            pl.BlockSpec((1, 1), lambda i: (0, 0)),
        ],
        out_specs=[
            pl.BlockSpec((1, 1, _BR), lambda i: (i, 0, 0)),
            pl.BlockSpec((_BR, LINE_), lambda i: (i, 0)),
        ],
        out_shape=[
            jax.ShapeDtypeStruct((_NBLK, 1, _BR), jnp.float32),
            jax.ShapeDtypeStruct((_NBLK * _BR, LINE_), jnp.float32),
        ],
    )(table_t, w1t, b1r, W2, b2r)
    return out.reshape(_NBLK * _BR), taug


_NW = 32
_RPT = BATCH_ // _NW
_SP = 208
_NL = 16
_NB = 2


def _pool_body(tok_hbm, mass_hbm, taug_hbm, out_hbm,
               tok_v, mb0, mb1, rows0, rows1, out_buf,
               sem_t, sem0, sem1):
    info = plsc.get_sparse_core_info()
    nc = info.num_cores
    wid = lax.axis_index("s") * nc + lax.axis_index("c")
    base_row = wid * _RPT
    rows = (rows0, rows1)
    mbs = (mb0, mb1)
    sems = (sem0, sem1)

    zf16 = jnp.zeros((_NL,), jnp.float32)

    for rbuf in rows:
        for s in range(SEQ_, _SP):
            for j in range(LINE_ // _NL):
                rbuf[s, pl.ds(j * _NL, _NL)] = zf16
    z16 = jnp.zeros((_NL,), jnp.int32)

    def zero_tok(r, _):
        tok_v[r, pl.ds(192, _NL)] = z16
        return _
    lax.fori_loop(0, _RPT, zero_tok, 0)

    pltpu.async_copy(tok_hbm.at[pl.ds(base_row, _RPT), :],
                     tok_v.at[:, pl.ds(0, SEQ_)], sem_t).wait()

    def fire_rows(r, slot, mb, sem):
        pltpu.async_copy(taug_hbm.at[tok_v.at[r, pl.ds(0, 112)]],
                         slot.at[pl.ds(0, 112)], sem)
        pltpu.async_copy(taug_hbm.at[tok_v.at[r, pl.ds(112, 88)]],
                         slot.at[pl.ds(112, 88)], sem)
        pltpu.async_copy(mass_hbm.at[tok_v.at[r, pl.ds(0, 112)]],
                         mb.at[pl.ds(0, 112)], sem)
        pltpu.async_copy(mass_hbm.at[tok_v.at[r, pl.ds(112, 88)]],
                         mb.at[pl.ds(112, 88)], sem)

    def wait_rows(r, slot, mb, sem):
        pltpu.make_async_copy(taug_hbm.at[tok_v.at[r, pl.ds(0, 112)]],
                              slot.at[pl.ds(0, 112)], sem).wait()
        pltpu.make_async_copy(taug_hbm.at[tok_v.at[r, pl.ds(112, 88)]],
                              slot.at[pl.ds(112, 88)], sem).wait()
        pltpu.make_async_copy(mass_hbm.at[tok_v.at[r, pl.ds(0, 112)]],
                              mb.at[pl.ds(0, 112)], sem).wait()
        pltpu.make_async_copy(mass_hbm.at[tok_v.at[r, pl.ds(112, 88)]],
                              mb.at[pl.ds(112, 88)], sem).wait()

    for b in range(_NB):
        fire_rows(b, rows[b], mbs[b], sems[b])

    def group_body(g, _):
        for b in range(_NB):
            r = g * _NB + b
            slot, mb, sem = rows[b], mbs[b], sems[b]
            wait_rows(r, slot, mb, sem)

            def acc_body(c, accs):
                a0, a1, a2, a3 = accs
                base = c * _NL
                tk = tok_v[r, pl.ds(base, _NL)]
                m = mb[pl.ds(base, _NL)]
                wv = jnp.where(tk == PAD_, zf16, jnp.exp(m))
                for k in range(_NL):
                    s = base + k
                    w = wv[k]
                    a0 = a0 + w * slot[s, pl.ds(0, _NL)]
                    a1 = a1 + w * slot[s, pl.ds(_NL, _NL)]
                    a2 = a2 + w * slot[s, pl.ds(2 * _NL, _NL)]
                    a3 = a3 + w * slot[s, pl.ds(3 * _NL, _NL)]
                return (a0, a1, a2, a3)

            a0, a1, a2, a3 = lax.fori_loop(0, 13, acc_body,
                                           (zf16, zf16, zf16, zf16))
            out_buf[r, pl.ds(0, _NL)] = a0
            out_buf[r, pl.ds(_NL, _NL)] = a1
            out_buf[r, pl.ds(2 * _NL, _NL)] = a2
            out_buf[r, pl.ds(3 * _NL, _NL)] = a3

            @pl.when(g < (_RPT // _NB) - 1)
            def _fire_next():
                fire_rows(r + _NB, slot, mb, sem)
        return _

    lax.fori_loop(0, _RPT // _NB, group_body, 0)
    pltpu.sync_copy(out_buf, out_hbm.at[pl.ds(base_row, _RPT)])


def _pool_sc(token_ids, mass, taug):
    mesh = plsc.VectorSubcoreMesh(core_axis_name="c", subcore_axis_name="s")
    f = functools.partial(
        pl.kernel,
        out_type=jax.ShapeDtypeStruct((BATCH_, DIM_), jnp.float32),
        mesh=mesh,
        scratch_types=[
            pltpu.VMEM((_RPT, _SP), jnp.int32),
            pltpu.VMEM((_SP,), jnp.float32),
            pltpu.VMEM((_SP,), jnp.float32),
            pltpu.VMEM((_SP, LINE_), jnp.float32),
            pltpu.VMEM((_SP, LINE_), jnp.float32),
            pltpu.VMEM((_RPT, DIM_), jnp.float32),
            pltpu.SemaphoreType.DMA,
            pltpu.SemaphoreType.DMA,
            pltpu.SemaphoreType.DMA,
        ],
        compiler_params=pltpu.CompilerParams(use_tc_tiling_on_sc=False),
    )(_pool_body)
    return f(token_ids, mass, taug)


def _norm_body(x_ref, o_ref):
    x = x_ref[...]
    n = jnp.sqrt(jnp.sum(x * x, axis=1, keepdims=True))
    o_ref[...] = x / jnp.maximum(n, 1e-12)


def _normalize_tc(sv):
    return pl.pallas_call(
        _norm_body,
        out_shape=jax.ShapeDtypeStruct((BATCH_, DIM_), jnp.float32),
    )(sv)


def kernel(token_ids, table, W1, b1, W2, b2):
    mass, taug = _mass_tc(table.T, W1, b1, W2, b2)
    sv = _pool_sc(token_ids, mass, taug)
    return _normalize_tc(sv)


# speedup vs baseline: 6.8525x; 1.0527x over previous
import functools

import jax
import jax.numpy as jnp
from jax import lax
from jax.experimental import pallas as pl
from jax.experimental.pallas import tpu as pltpu
from jax.experimental.pallas import tpu_sc as plsc

VOCAB_ = 1000000
DIM_ = 64
HID_ = 16
PAD_ = 0
BATCH_ = 4096
SEQ_ = 200
LINE_ = 128


_BR = 16384
_NBLK = (VOCAB_ + _BR - 1) // _BR


def _mass_body(t_ref, w1t_ref, b1_ref, w2_ref, b2_ref, o_ref, o2_ref):
    t = t_ref[...]
    x = jnp.dot(w1t_ref[...], t, preferred_element_type=jnp.float32)
    x = x + b1_ref[...]
    e2 = jnp.exp(2.0 * x)
    h = 1.0 - 2.0 / (e2 + 1.0)
    raw = jnp.sum(h * w2_ref[...], axis=0) + b2_ref[0, 0]
    o_ref[...] = raw.reshape(1, 1, _BR)
    o2_ref[:, :DIM_] = t.T
    o2_ref[:, DIM_:] = jnp.zeros((_BR, LINE_ - DIM_), jnp.float32)


def _mass_tc(table_t, W1, b1, W2, b2):
    w1t = W1.T
    b1r = b1.reshape(HID_, 1)
    b2r = b2.reshape(1, 1)
    out, taug = pl.pallas_call(
        _mass_body,
        grid=(_NBLK,),
        in_specs=[
            pl.BlockSpec((DIM_, _BR), lambda i: (0, i)),
            pl.BlockSpec((HID_, DIM_), lambda i: (0, 0)),
            pl.BlockSpec((HID_, 1), lambda i: (0, 0)),
            pl.BlockSpec((HID_, 1), lambda i: (0, 0)),
            pl.BlockSpec((1, 1), lambda i: (0, 0)),
        ],
        out_specs=[
            pl.BlockSpec((1, 1, _BR), lambda i: (i, 0, 0)),
            pl.BlockSpec((_BR, LINE_), lambda i: (i, 0)),
        ],
        out_shape=[
            jax.ShapeDtypeStruct((_NBLK, 1, _BR), jnp.float32),
            jax.ShapeDtypeStruct((_NBLK * _BR, LINE_), jnp.float32),
        ],
    )(table_t, w1t, b1r, W2, b2r)
    return out.reshape(_NBLK * _BR), taug


_NW = 32
_RPT = BATCH_ // _NW
_SP = 208
_NL = 16
_NB = 3


def _pool_body(tok_hbm, mass_hbm, taug_hbm, out_hbm,
               tok_v, mb0, mb1, mb2, rows0, rows1, rows2, out_buf,
               sem_t, sem0, sem1, sem2):
    info = plsc.get_sparse_core_info()
    nc = info.num_cores
    wid = lax.axis_index("s") * nc + lax.axis_index("c")
    base_row = wid * _RPT
    rows = (rows0, rows1, rows2)
    mbs = (mb0, mb1, mb2)
    sems = (sem0, sem1, sem2)

    zf16 = jnp.zeros((_NL,), jnp.float32)

    for rbuf in rows:
        for s in range(SEQ_, _SP):
            for j in range(LINE_ // _NL):
                rbuf[s, pl.ds(j * _NL, _NL)] = zf16
    z16 = jnp.zeros((_NL,), jnp.int32)

    def zero_tok(r, _):
        tok_v[r, pl.ds(192, _NL)] = z16
        return _
    lax.fori_loop(0, _RPT, zero_tok, 0)

    pltpu.async_copy(tok_hbm.at[pl.ds(base_row, _RPT), :],
                     tok_v.at[:, pl.ds(0, SEQ_)], sem_t).wait()

    def fire_rows(r, slot, mb, sem):
        pltpu.async_copy(taug_hbm.at[tok_v.at[r, pl.ds(0, 112)]],
                         slot.at[pl.ds(0, 112)], sem)
        pltpu.async_copy(taug_hbm.at[tok_v.at[r, pl.ds(112, 88)]],
                         slot.at[pl.ds(112, 88)], sem)
        pltpu.async_copy(mass_hbm.at[tok_v.at[r, pl.ds(0, 112)]],
                         mb.at[pl.ds(0, 112)], sem)
        pltpu.async_copy(mass_hbm.at[tok_v.at[r, pl.ds(112, 88)]],
                         mb.at[pl.ds(112, 88)], sem)

    def wait_rows(r, slot, mb, sem):
        pltpu.make_async_copy(taug_hbm.at[tok_v.at[r, pl.ds(0, 112)]],
                              slot.at[pl.ds(0, 112)], sem).wait()
        pltpu.make_async_copy(taug_hbm.at[tok_v.at[r, pl.ds(112, 88)]],
                              slot.at[pl.ds(112, 88)], sem).wait()
        pltpu.make_async_copy(mass_hbm.at[tok_v.at[r, pl.ds(0, 112)]],
                              mb.at[pl.ds(0, 112)], sem).wait()
        pltpu.make_async_copy(mass_hbm.at[tok_v.at[r, pl.ds(112, 88)]],
                              mb.at[pl.ds(112, 88)], sem).wait()

    for b in range(_NB):
        fire_rows(b, rows[b], mbs[b], sems[b])

    def compute_row(r, slot, mb, sem):
        wait_rows(r, slot, mb, sem)

        def acc_body(c, accs):
            a0, a1, a2, a3 = accs
            base = c * _NL
            tk = tok_v[r, pl.ds(base, _NL)]
            m = mb[pl.ds(base, _NL)]
            wv = jnp.where(tk == PAD_, zf16, jnp.exp(m))
            for k in range(_NL):
                s = base + k
                w = wv[k]
                a0 = a0 + w * slot[s, pl.ds(0, _NL)]
                a1 = a1 + w * slot[s, pl.ds(_NL, _NL)]
                a2 = a2 + w * slot[s, pl.ds(2 * _NL, _NL)]
                a3 = a3 + w * slot[s, pl.ds(3 * _NL, _NL)]
            return (a0, a1, a2, a3)

        a0, a1, a2, a3 = lax.fori_loop(0, 13, acc_body,
                                       (zf16, zf16, zf16, zf16))
        out_buf[r, pl.ds(0, _NL)] = a0
        out_buf[r, pl.ds(_NL, _NL)] = a1
        out_buf[r, pl.ds(2 * _NL, _NL)] = a2
        out_buf[r, pl.ds(3 * _NL, _NL)] = a3

    _NG = _RPT // _NB

    def group_body(g, _):
        for b in range(_NB):
            r = g * _NB + b
            slot, mb, sem = rows[b], mbs[b], sems[b]
            compute_row(r, slot, mb, sem)

            @pl.when(r + _NB < _RPT)
            def _fire_next():
                fire_rows(r + _NB, slot, mb, sem)
        return _

    lax.fori_loop(0, _NG, group_body, 0)
    for r in range(_NG * _NB, _RPT):
        b = r % _NB
        compute_row(r, rows[b], mbs[b], sems[b])
    pltpu.sync_copy(out_buf, out_hbm.at[pl.ds(base_row, _RPT)])


def _pool_sc(token_ids, mass, taug):
    mesh = plsc.VectorSubcoreMesh(core_axis_name="c", subcore_axis_name="s")
    f = functools.partial(
        pl.kernel,
        out_type=jax.ShapeDtypeStruct((BATCH_, DIM_), jnp.float32),
        mesh=mesh,
        scratch_types=[
            pltpu.VMEM((_RPT, _SP), jnp.int32),
            pltpu.VMEM((_SP,), jnp.float32),
            pltpu.VMEM((_SP,), jnp.float32),
            pltpu.VMEM((_SP,), jnp.float32),
            pltpu.VMEM((_SP, LINE_), jnp.float32),
            pltpu.VMEM((_SP, LINE_), jnp.float32),
            pltpu.VMEM((_SP, LINE_), jnp.float32),
            pltpu.VMEM((_RPT, DIM_), jnp.float32),
            pltpu.SemaphoreType.DMA,
            pltpu.SemaphoreType.DMA,
            pltpu.SemaphoreType.DMA,
            pltpu.SemaphoreType.DMA,
        ],
        compiler_params=pltpu.CompilerParams(use_tc_tiling_on_sc=False),
    )(_pool_body)
    return f(token_ids, mass, taug)


def _norm_body(x_ref, o_ref):
    x = x_ref[...]
    n = jnp.sqrt(jnp.sum(x * x, axis=1, keepdims=True))
    o_ref[...] = x / jnp.maximum(n, 1e-12)


def _normalize_tc(sv):
    return pl.pallas_call(
        _norm_body,
        out_shape=jax.ShapeDtypeStruct((BATCH_, DIM_), jnp.float32),
    )(sv)


def kernel(token_ids, table, W1, b1, W2, b2):
    mass, taug = _mass_tc(table.T, W1, b1, W2, b2)
    sv = _pool_sc(token_ids, mass, taug)
    return _normalize_tc(sv)
